# baseline (speedup 1.0000x reference)
import functools

import jax
import jax.numpy as jnp
from jax import lax
from jax.experimental import pallas as pl
from jax.experimental.pallas import tpu as pltpu
from jax.experimental.pallas import tpu_sc as plsc

_N = 10000
_M = 32
_D = 128
_E = 16
_CH = 128
_NW = 32
_NBUF = 4
_NH = _N // 2
_NCH = (_NH * _M) // _CH
_CPW = 40
_BI = 200
_GRID = _NH // _BI
_EPS = 1e-5


def _softplus(x):
    return jnp.maximum(x, 0.0) + jnp.log1p(jnp.exp(-jnp.abs(x)))


def _dott(a, b):
    return lax.dot_general(a, b, (((0,), (0,)), ((), ())),
                           preferred_element_type=jnp.float32)


def _sc_gather_body(table_hbm, idx_hbm, out_hbm, idx_v, rows_v, gsem, wsem):
    wid = lax.axis_index("s") * 2 + lax.axis_index("c")
    base = wid * _CPW
    pltpu.sync_copy(idx_hbm.at[pl.ds(base * _CH, _CPW * _CH)], idx_v)

    def _wb_drain():
        pltpu.make_async_copy(
            rows_v.at[0], out_hbm.at[pl.ds(0, _CH)], wsem).wait()

    def _gather_fire(j, b):
        pltpu.async_copy(
            table_hbm.at[idx_v.at[pl.ds(j * _CH, _CH)]], rows_v.at[b], gsem)

    def _gather_wait(j, b):
        pltpu.make_async_copy(
            table_hbm.at[idx_v.at[pl.ds(j * _CH, _CH)]], rows_v.at[b],
            gsem).wait()

    def _wb_fire(b, c):
        pltpu.async_copy(rows_v.at[b], out_hbm.at[pl.ds(c * _CH, _CH)], wsem)

    ngrp = _CPW // _NBUF

    def body(g, carry):
        for b in range(_NBUF):
            j = g * _NBUF + b
            c = base + j

            @pl.when((g > 0) & (c - _NBUF < _NCH))
            def _():
                _wb_drain()

            @pl.when(c < _NCH)
            def _():
                _gather_fire(j, b)

            prev_ok = (c - 1 < _NCH)
            if b == 0:
                prev_ok = (g > 0) & prev_ok

            @pl.when(prev_ok)
            def _():
                bp = (b - 1) % _NBUF
                _gather_wait(j - 1, bp)
                _wb_fire(bp, c - 1)

        return carry

    lax.fori_loop(0, ngrp, body, 0)

    last = _CPW - 1

    @pl.when(base + last < _NCH)
    def _():
        _gather_wait(last, last % _NBUF)
        _wb_fire(last % _NBUF, base + last)

    for j in range(_CPW - _NBUF, _CPW):
        @pl.when(base + j < _NCH)
        def _():
            _wb_drain()


def _sc_gather(table, idxp):
    call = pl.kernel(
        _sc_gather_body,
        out_type=jax.ShapeDtypeStruct((_NH * _M, _D), jnp.float32),
        mesh=plsc.VectorSubcoreMesh(core_axis_name="c", subcore_axis_name="s"),
        scratch_types=[
            pltpu.VMEM((_CPW * _CH,), jnp.int32),
            pltpu.VMEM((_NBUF, _CH, _D), jnp.float32),
            pltpu.SemaphoreType.DMA,
            pltpu.SemaphoreType.DMA,
        ],
    )
    return call(table, idxp)


def _moments_body(atom_ref, ag_ref, nbr_ref, wself_ref, we_ref, wnbr_ref,
                  b_ref, out_ref, yy, ab, zz):
    i = pl.program_id(0)

    @pl.when(i == 0)
    def _():
        out_ref[...] = jnp.zeros_like(out_ref)
        yy[...] = jnp.zeros_like(yy)
        ab[...] = jnp.zeros_like(ab)
        zz[...] = jnp.zeros_like(zz)

    ag = ag_ref[...]
    nbr3 = nbr_ref[...]
    nbrf = nbr3.reshape(_BI * _M, _E)

    yy[...] += _dott(ag, ag)
    ab[...] += _dott(nbrf, ag)
    zz[...] += _dott(nbrf, nbrf)

    agsum = jnp.sum(ag.reshape(_BI, _M, _D), axis=1)
    nbrsum = jnp.sum(nbr3, axis=1)

    sp = jnp.dot(atom_ref[...], wself_ref[...],
                 preferred_element_type=jnp.float32) + b_ref[...]
    q = jnp.dot(agsum, wnbr_ref[...], preferred_element_type=jnp.float32)
    esum = jnp.dot(nbrsum, we_ref[...], preferred_element_type=jnp.float32)

    mf = jnp.float32(_M)
    t1 = jnp.sum(mf * sp + esum + q, axis=0)
    t2 = jnp.sum(mf * sp * sp + 2.0 * sp * (esum + q), axis=0)
    out_ref[0:1, :] += t1[None, :]
    out_ref[1:2, :] += t2[None, :]

    @pl.when(i == pl.num_programs(0) - 1)
    def _():
        wnbr = wnbr_ref[...]
        we = we_ref[...]
        g2 = jnp.sum(wnbr * jnp.dot(yy[...], wnbr,
                                    preferred_element_type=jnp.float32), 0)
        e2 = jnp.sum(we * jnp.dot(zz[...], we,
                                  preferred_element_type=jnp.float32), 0)
        eg = jnp.sum(we * jnp.dot(ab[...], wnbr,
                                  preferred_element_type=jnp.float32), 0)
        out_ref[1:2, :] += (g2 + e2 + 2.0 * eg)[None, :]


def _reduce_body(atom_ref, ag_ref, nbr_ref, wselfs_ref, wes_ref, wnbrs_ref,
                 bs_ref, ns_ref, st2_ref):
    s = jnp.dot(atom_ref[...], wselfs_ref[...],
                preferred_element_type=jnp.float32) + bs_ref[...]
    g = jnp.dot(ag_ref[...], wnbrs_ref[...], preferred_element_type=jnp.float32)
    e = jnp.dot(nbr_ref[...].reshape(_BI * _M, _E), wes_ref[...],
                preferred_element_type=jnp.float32)
    h = (g + e).reshape(_BI, _M, 2 * _D) + s[:, None, :]
    filt = jax.nn.sigmoid(h[:, :, :_D])
    core = _softplus(h[:, :, _D:])
    ns = jnp.sum(filt * core, axis=1)
    ns_ref[...] = ns

    @pl.when(pl.program_id(0) == 0)
    def _():
        st2_ref[...] = jnp.zeros_like(st2_ref)

    st2_ref[0:1, :] += jnp.sum(ns, axis=0)[None, :]
    st2_ref[1:2, :] += jnp.sum(ns * ns, axis=0)[None, :]


def _final_body(atom_ref, ns_ref, wskip_ref, bskip_ref, a2_ref, b2_ref,
                out_ref):
    skip = jnp.dot(atom_ref[...], wskip_ref[...],
                   preferred_element_type=jnp.float32) + bskip_ref[...]
    out_ref[...] = _softplus(skip + ns_ref[...] * a2_ref[...] + b2_ref[...])


def _half_specs(off):
    return [
        pl.BlockSpec((_BI, _D), lambda i: (i + off, 0)),
        pl.BlockSpec((_BI * _M, _D), lambda i: (i, 0)),
        pl.BlockSpec((_BI, _M, _E), lambda i: (i + off, 0, 0)),
        pl.BlockSpec((_D, 2 * _D), lambda i: (0, 0)),
        pl.BlockSpec((_E, 2 * _D), lambda i: (0, 0)),
        pl.BlockSpec((_D, 2 * _D), lambda i: (0, 0)),
        pl.BlockSpec((1, 2 * _D), lambda i: (0, 0)),
    ]


def kernel(atom_in_fea, nbr_fea, nbr_fea_idx, W_full, b_full, bn1_gamma,
           bn1_beta, bn2_gamma, bn2_beta, W_skip, b_skip):
    idxf = nbr_fea_idx.astype(jnp.int32).reshape(_N * _M)
    pad = _NW * _CPW * _CH - _NH * _M
    idxp1 = jnp.pad(idxf[:_NH * _M], (0, pad))
    idxp2 = jnp.pad(idxf[_NH * _M:], (0, pad))

    w_self = W_full[:_D]
    w_nbr = W_full[_D:2 * _D]
    w_e = W_full[2 * _D:]
    b2d = b_full.reshape(1, 2 * _D)

    ag1 = _sc_gather(atom_in_fea, idxp1)
    ag2 = _sc_gather(atom_in_fea, idxp2)

    def _moments(ag, off):
        return pl.pallas_call(
            _moments_body,
            grid=(_GRID,),
            in_specs=_half_specs(off),
            out_specs=pl.BlockSpec((8, 2 * _D), lambda i: (0, 0)),
            out_shape=jax.ShapeDtypeStruct((8, 2 * _D), jnp.float32),
            scratch_shapes=[
                pltpu.VMEM((_D, _D), jnp.float32),
                pltpu.VMEM((_E, _D), jnp.float32),
                pltpu.VMEM((_E, _E), jnp.float32),
            ],
        )(atom_in_fea, ag, nbr_fea, w_self, w_e, w_nbr, b2d)

    st1a = _moments(ag1, 0)
    st1b = _moments(ag2, _GRID)
    stats1 = st1a + st1b

    cnt1 = jnp.float32(_N * _M)
    mean1 = stats1[0] / cnt1
    var1 = stats1[1] / cnt1 - mean1 * mean1
    a1 = bn1_gamma / jnp.sqrt(var1 + _EPS)
    b1 = bn1_beta - mean1 * a1

    wselfs = w_self * a1
    wes = w_e * a1
    wnbrs = w_nbr * a1
    bs = (b_full * a1 + b1).reshape(1, 2 * _D)

    def _reduce(ag, off):
        return pl.pallas_call(
            _reduce_body,
            grid=(_GRID,),
            in_specs=_half_specs(off),
            out_specs=[
                pl.BlockSpec((_BI, _D), lambda i: (i, 0)),
                pl.BlockSpec((8, _D), lambda i: (0, 0)),
            ],
            out_shape=[
                jax.ShapeDtypeStruct((_NH, _D), jnp.float32),
                jax.ShapeDtypeStruct((8, _D), jnp.float32),
            ],
        )(atom_in_fea, ag, nbr_fea, wselfs, wes, wnbrs, bs)

    ns1, st2a = _reduce(ag1, 0)
    ns2, st2b = _reduce(ag2, _GRID)
    stats2 = st2a + st2b

    cnt2 = jnp.float32(_N)
    mean2 = stats2[0] / cnt2
    var2 = stats2[1] / cnt2 - mean2 * mean2
    a2 = bn2_gamma / jnp.sqrt(var2 + _EPS)
    b2 = bn2_beta - mean2 * a2

    ns = jnp.concatenate([ns1, ns2], axis=0)

    out = pl.pallas_call(
        _final_body,
        grid=(2 * _GRID,),
        in_specs=[
            pl.BlockSpec((_BI, _D), lambda i: (i, 0)),
            pl.BlockSpec((_BI, _D), lambda i: (i, 0)),
            pl.BlockSpec((_D, _D), lambda i: (0, 0)),
            pl.BlockSpec((1, _D), lambda i: (0, 0)),
            pl.BlockSpec((1, _D), lambda i: (0, 0)),
            pl.BlockSpec((1, _D), lambda i: (0, 0)),
        ],
        out_specs=pl.BlockSpec((_BI, _D), lambda i: (i, 0)),
        out_shape=jax.ShapeDtypeStruct((_N, _D), jnp.float32),
    )(atom_in_fea, ns, W_skip, b_skip.reshape(1, -1),
      a2.reshape(1, -1), b2.reshape(1, -1))

    return out


# speedup vs baseline: 1.0799x; 1.0799x over previous
import functools

import jax
import jax.numpy as jnp
from jax import lax
from jax.experimental import pallas as pl
from jax.experimental.pallas import tpu as pltpu
from jax.experimental.pallas import tpu_sc as plsc

_N = 10000
_M = 32
_D = 128
_E = 16
_CH = 128
_NCHUNK = (_N * _M) // _CH
_NW = 32
_CPW = 80
_NBUF = 4
_BI = 400
_GRID = _N // _BI
_EPS = 1e-5


def _softplus(x):
    return jnp.maximum(x, 0.0) + jnp.log1p(jnp.exp(-jnp.abs(x)))


def _dott(a, b):
    return lax.dot_general(a, b, (((0,), (0,)), ((), ())),
                           preferred_element_type=jnp.float32)


def _sc_gather_body(table_hbm, idx_hbm, out_hbm, idx_v, rows_v, gsem, wsem):
    wid = lax.axis_index("s") * 2 + lax.axis_index("c")
    base = wid * _CPW
    pltpu.sync_copy(idx_hbm.at[pl.ds(base * _CH, _CPW * _CH)], idx_v)

    def _wb_drain():
        pltpu.make_async_copy(
            rows_v.at[0], out_hbm.at[pl.ds(0, _CH)], wsem).wait()

    def _gather_fire(j, b):
        pltpu.async_copy(
            table_hbm.at[idx_v.at[pl.ds(j * _CH, _CH)]], rows_v.at[b], gsem)

    def _gather_wait(j, b):
        pltpu.make_async_copy(
            table_hbm.at[idx_v.at[pl.ds(j * _CH, _CH)]], rows_v.at[b],
            gsem).wait()

    def _wb_fire(b, c):
        pltpu.async_copy(rows_v.at[b], out_hbm.at[pl.ds(c * _CH, _CH)], wsem)

    ngrp = _CPW // _NBUF

    def body(g, carry):
        for b in range(_NBUF):
            j = g * _NBUF + b
            c = base + j

            @pl.when((g > 0) & (c - _NBUF < _NCHUNK))
            def _():
                _wb_drain()

            @pl.when(c < _NCHUNK)
            def _():
                _gather_fire(j, b)

            prev_ok = (c - 1 < _NCHUNK)
            if b == 0:
                prev_ok = (g > 0) & prev_ok

            @pl.when(prev_ok)
            def _():
                bp = (b - 1) % _NBUF
                _gather_wait(j - 1, bp)
                _wb_fire(bp, c - 1)

        return carry

    lax.fori_loop(0, ngrp, body, 0)

    last = _CPW - 1

    @pl.when(base + last < _NCHUNK)
    def _():
        _gather_wait(last, last % _NBUF)
        _wb_fire(last % _NBUF, base + last)

    for j in range(_CPW - _NBUF, _CPW):
        @pl.when(base + j < _NCHUNK)
        def _():
            _wb_drain()


def _sc_gather(table, idxp):
    call = pl.kernel(
        _sc_gather_body,
        out_type=jax.ShapeDtypeStruct((_N * _M, _D), jnp.float32),
        mesh=plsc.VectorSubcoreMesh(core_axis_name="c", subcore_axis_name="s"),
        scratch_types=[
            pltpu.VMEM((_CPW * _CH,), jnp.int32),
            pltpu.VMEM((_NBUF, _CH, _D), jnp.float32),
            pltpu.SemaphoreType.DMA,
            pltpu.SemaphoreType.DMA,
        ],
    )
    return call(table, idxp)


def _moments_body(atom_ref, ag_ref, nbr_ref, wself_ref, we_ref, wnbr_ref,
                  b_ref, out_ref, yy, ab, zz):
    i = pl.program_id(0)

    @pl.when(i == 0)
    def _():
        out_ref[...] = jnp.zeros_like(out_ref)
        yy[...] = jnp.zeros_like(yy)
        ab[...] = jnp.zeros_like(ab)
        zz[...] = jnp.zeros_like(zz)

    ag = ag_ref[...]
    nbr3 = nbr_ref[...]
    nbrf = nbr3.reshape(_BI * _M, _E)

    yy[...] += _dott(ag, ag)
    ab[...] += _dott(nbrf, ag)
    zz[...] += _dott(nbrf, nbrf)

    agsum = jnp.sum(ag.reshape(_BI, _M, _D), axis=1)
    nbrsum = jnp.sum(nbr3, axis=1)

    sp = jnp.dot(atom_ref[...], wself_ref[...],
                 preferred_element_type=jnp.float32) + b_ref[...]
    q = jnp.dot(agsum, wnbr_ref[...], preferred_element_type=jnp.float32)
    esum = jnp.dot(nbrsum, we_ref[...], preferred_element_type=jnp.float32)

    mf = jnp.float32(_M)
    t1 = jnp.sum(mf * sp + esum + q, axis=0)
    t2 = jnp.sum(mf * sp * sp + 2.0 * sp * (esum + q), axis=0)
    out_ref[0:1, :] += t1[None, :]
    out_ref[1:2, :] += t2[None, :]

    @pl.when(i == _GRID - 1)
    def _():
        wnbr = wnbr_ref[...]
        we = we_ref[...]
        g2 = jnp.sum(wnbr * jnp.dot(yy[...], wnbr,
                                    preferred_element_type=jnp.float32), 0)
        e2 = jnp.sum(we * jnp.dot(zz[...], we,
                                  preferred_element_type=jnp.float32), 0)
        eg = jnp.sum(we * jnp.dot(ab[...], wnbr,
                                  preferred_element_type=jnp.float32), 0)
        out_ref[1:2, :] += (g2 + e2 + 2.0 * eg)[None, :]


def _reduce_body(atom_ref, ag_ref, nbr_ref, wselfs_ref, wes_ref, wnbrs_ref,
                 bs_ref, ns_ref, st2_ref):
    s = jnp.dot(atom_ref[...], wselfs_ref[...],
                preferred_element_type=jnp.float32) + bs_ref[...]
    g = jnp.dot(ag_ref[...], wnbrs_ref[...], preferred_element_type=jnp.float32)
    e = jnp.dot(nbr_ref[...].reshape(_BI * _M, _E), wes_ref[...],
                preferred_element_type=jnp.float32)
    h = (g + e).reshape(_BI, _M, 2 * _D) + s[:, None, :]
    filt = jax.nn.sigmoid(h[:, :, :_D])
    core = _softplus(h[:, :, _D:])
    ns = jnp.sum(filt * core, axis=1)
    ns_ref[...] = ns

    @pl.when(pl.program_id(0) == 0)
    def _():
        st2_ref[...] = jnp.zeros_like(st2_ref)

    st2_ref[0:1, :] += jnp.sum(ns, axis=0)[None, :]
    st2_ref[1:2, :] += jnp.sum(ns * ns, axis=0)[None, :]


def _final_body(atom_ref, ns_ref, wskip_ref, bskip_ref, a2_ref, b2_ref,
                out_ref):
    skip = jnp.dot(atom_ref[...], wskip_ref[...],
                   preferred_element_type=jnp.float32) + bskip_ref[...]
    out_ref[...] = _softplus(skip + ns_ref[...] * a2_ref[...] + b2_ref[...])


def kernel(atom_in_fea, nbr_fea, nbr_fea_idx, W_full, b_full, bn1_gamma,
           bn1_beta, bn2_gamma, bn2_beta, W_skip, b_skip):
    idxf = nbr_fea_idx.astype(jnp.int32).reshape(_N * _M)
    idxp = jnp.pad(idxf, (0, (_NW * _CPW - _NCHUNK) * _CH))

    w_self = W_full[:_D]
    w_nbr = W_full[_D:2 * _D]
    w_e = W_full[2 * _D:]
    b2d = b_full.reshape(1, 2 * _D)

    ag = _sc_gather(atom_in_fea, idxp)

    wspec = [
        pl.BlockSpec((_D, 2 * _D), lambda i: (0, 0)),
        pl.BlockSpec((_E, 2 * _D), lambda i: (0, 0)),
        pl.BlockSpec((_D, 2 * _D), lambda i: (0, 0)),
        pl.BlockSpec((1, 2 * _D), lambda i: (0, 0)),
    ]
    row_specs = [
        pl.BlockSpec((_BI, _D), lambda i: (i, 0)),
        pl.BlockSpec((_BI * _M, _D), lambda i: (i, 0)),
        pl.BlockSpec((_BI, _M, _E), lambda i: (i, 0, 0)),
    ] + wspec

    stats1 = pl.pallas_call(
        _moments_body,
        grid=(_GRID,),
        in_specs=row_specs,
        out_specs=pl.BlockSpec((8, 2 * _D), lambda i: (0, 0)),
        out_shape=jax.ShapeDtypeStruct((8, 2 * _D), jnp.float32),
        scratch_shapes=[
            pltpu.VMEM((_D, _D), jnp.float32),
            pltpu.VMEM((_E, _D), jnp.float32),
            pltpu.VMEM((_E, _E), jnp.float32),
        ],
    )(atom_in_fea, ag, nbr_fea, w_self, w_e, w_nbr, b2d)

    cnt1 = jnp.float32(_N * _M)
    mean1 = stats1[0] / cnt1
    var1 = stats1[1] / cnt1 - mean1 * mean1
    a1 = bn1_gamma / jnp.sqrt(var1 + _EPS)
    b1 = bn1_beta - mean1 * a1

    ns, stats2 = pl.pallas_call(
        _reduce_body,
        grid=(_GRID,),
        in_specs=row_specs,
        out_specs=[
            pl.BlockSpec((_BI, _D), lambda i: (i, 0)),
            pl.BlockSpec((8, _D), lambda i: (0, 0)),
        ],
        out_shape=[
            jax.ShapeDtypeStruct((_N, _D), jnp.float32),
            jax.ShapeDtypeStruct((8, _D), jnp.float32),
        ],
    )(atom_in_fea, ag, nbr_fea, w_self * a1, w_e * a1, w_nbr * a1,
      (b_full * a1 + b1).reshape(1, 2 * _D))

    cnt2 = jnp.float32(_N)
    mean2 = stats2[0] / cnt2
    var2 = stats2[1] / cnt2 - mean2 * mean2
    a2 = bn2_gamma / jnp.sqrt(var2 + _EPS)
    b2 = bn2_beta - mean2 * a2

    out = pl.pallas_call(
        _final_body,
        grid=(_GRID,),
        in_specs=[
            pl.BlockSpec((_BI, _D), lambda i: (i, 0)),
            pl.BlockSpec((_BI, _D), lambda i: (i, 0)),
            pl.BlockSpec((_D, _D), lambda i: (0, 0)),
            pl.BlockSpec((1, _D), lambda i: (0, 0)),
            pl.BlockSpec((1, _D), lambda i: (0, 0)),
            pl.BlockSpec((1, _D), lambda i: (0, 0)),
        ],
        out_specs=pl.BlockSpec((_BI, _D), lambda i: (i, 0)),
        out_shape=jax.ShapeDtypeStruct((_N, _D), jnp.float32),
    )(atom_in_fea, ns, W_skip, b_skip.reshape(1, -1),
      a2.reshape(1, -1), b2.reshape(1, -1))

    return out


# speedup vs baseline: 1.0942x; 1.0132x over previous
import functools

import jax
import jax.numpy as jnp
from jax import lax
from jax.experimental import pallas as pl
from jax.experimental.pallas import tpu as pltpu
from jax.experimental.pallas import tpu_sc as plsc

_N = 10000
_M = 32
_D = 128
_E = 16
_CH = 128
_NCHUNK = (_N * _M) // _CH
_NW = 32
_CPW = 80
_NBUF = 4
_BI = 400
_GRID = _N // _BI
_EPS = 1e-5


def _softplus(x):
    return jnp.maximum(x, 0.0) + jnp.log1p(jnp.exp(-jnp.abs(x)))


def _dott(a, b):
    return lax.dot_general(a, b, (((0,), (0,)), ((), ())),
                           preferred_element_type=jnp.float32)


def _sc_gather_body(table_hbm, idx_hbm, out_hbm, idx_v, rows_v, gsem, wsem):
    wid = lax.axis_index("s") * 2 + lax.axis_index("c")
    base = wid * _CPW
    pltpu.sync_copy(idx_hbm.at[pl.ds(base * _CH, _CPW * _CH)], idx_v)

    def _wb_drain():
        pltpu.make_async_copy(
            rows_v.at[0], out_hbm.at[pl.ds(0, _CH)], wsem).wait()

    def _gather_fire(j, b):
        pltpu.async_copy(
            table_hbm.at[idx_v.at[pl.ds(j * _CH, _CH)]], rows_v.at[b], gsem)

    def _gather_wait(j, b):
        pltpu.make_async_copy(
            table_hbm.at[idx_v.at[pl.ds(j * _CH, _CH)]], rows_v.at[b],
            gsem).wait()

    def _wb_fire(b, c):
        pltpu.async_copy(rows_v.at[b], out_hbm.at[pl.ds(c * _CH, _CH)], wsem)

    ngrp = _CPW // _NBUF

    def body(g, carry):
        for b in range(_NBUF):
            j = g * _NBUF + b
            c = base + j

            @pl.when((g > 0) & (c - _NBUF < _NCHUNK))
            def _():
                _wb_drain()

            @pl.when(c < _NCHUNK)
            def _():
                _gather_fire(j, b)

            prev_ok = (c - 1 < _NCHUNK)
            if b == 0:
                prev_ok = (g > 0) & prev_ok

            @pl.when(prev_ok)
            def _():
                bp = (b - 1) % _NBUF
                _gather_wait(j - 1, bp)
                _wb_fire(bp, c - 1)

        return carry

    lax.fori_loop(0, ngrp, body, 0)

    last = _CPW - 1

    @pl.when(base + last < _NCHUNK)
    def _():
        _gather_wait(last, last % _NBUF)
        _wb_fire(last % _NBUF, base + last)

    for j in range(_CPW - _NBUF, _CPW):
        @pl.when(base + j < _NCHUNK)
        def _():
            _wb_drain()


def _sc_gather(table, idxp):
    call = pl.kernel(
        _sc_gather_body,
        out_type=jax.ShapeDtypeStruct((_N * _M, _D), jnp.float32),
        mesh=plsc.VectorSubcoreMesh(core_axis_name="c", subcore_axis_name="s"),
        scratch_types=[
            pltpu.VMEM((_CPW * _CH,), jnp.int32),
            pltpu.VMEM((_NBUF, _CH, _D), jnp.float32),
            pltpu.SemaphoreType.DMA,
            pltpu.SemaphoreType.DMA,
        ],
    )
    return call(table, idxp)


def _moments_body(atom_ref, ag_ref, nbr_ref, wself_ref, we_ref, wnbr_ref,
                  b_ref, out_ref, yy, ab, zz):
    i = pl.program_id(0)

    @pl.when(i == 0)
    def _():
        out_ref[...] = jnp.zeros_like(out_ref)
        yy[...] = jnp.zeros_like(yy)
        ab[...] = jnp.zeros_like(ab)
        zz[...] = jnp.zeros_like(zz)

    ag = ag_ref[...]
    nbr3 = nbr_ref[...]
    nbrf = nbr3.reshape(_BI * _M, _E)

    agb = ag.astype(jnp.bfloat16)
    nbrb = nbrf.astype(jnp.bfloat16)
    yy[...] += _dott(agb, agb)
    ab[...] += _dott(nbrb, agb)
    zz[...] += _dott(nbrb, nbrb)

    agsum = jnp.sum(ag.reshape(_BI, _M, _D), axis=1)
    nbrsum = jnp.sum(nbr3, axis=1)

    sp = jnp.dot(atom_ref[...], wself_ref[...],
                 preferred_element_type=jnp.float32) + b_ref[...]
    q = jnp.dot(agsum, wnbr_ref[...], preferred_element_type=jnp.float32)
    esum = jnp.dot(nbrsum, we_ref[...], preferred_element_type=jnp.float32)

    mf = jnp.float32(_M)
    t1 = jnp.sum(mf * sp + esum + q, axis=0)
    t2 = jnp.sum(mf * sp * sp + 2.0 * sp * (esum + q), axis=0)
    out_ref[0:1, :] += t1[None, :]
    out_ref[1:2, :] += t2[None, :]

    @pl.when(i == _GRID - 1)
    def _():
        wnbr = wnbr_ref[...]
        we = we_ref[...]
        g2 = jnp.sum(wnbr * jnp.dot(yy[...], wnbr,
                                    preferred_element_type=jnp.float32), 0)
        e2 = jnp.sum(we * jnp.dot(zz[...], we,
                                  preferred_element_type=jnp.float32), 0)
        eg = jnp.sum(we * jnp.dot(ab[...], wnbr,
                                  preferred_element_type=jnp.float32), 0)
        out_ref[1:2, :] += (g2 + e2 + 2.0 * eg)[None, :]


def _reduce_body(atom_ref, ag_ref, nbr_ref, wselfs_ref, wes_ref, wnbrs_ref,
                 bs_ref, ns_ref, st2_ref):
    s = jnp.dot(atom_ref[...], wselfs_ref[...],
                preferred_element_type=jnp.float32) + bs_ref[...]
    g = jnp.dot(ag_ref[...], wnbrs_ref[...], preferred_element_type=jnp.float32)
    e = jnp.dot(nbr_ref[...].reshape(_BI * _M, _E), wes_ref[...],
                preferred_element_type=jnp.float32)
    h = (g + e).reshape(_BI, _M, 2 * _D) + s[:, None, :]
    filt = jax.nn.sigmoid(h[:, :, :_D])
    core = _softplus(h[:, :, _D:])
    ns = jnp.sum(filt * core, axis=1)
    ns_ref[...] = ns

    @pl.when(pl.program_id(0) == 0)
    def _():
        st2_ref[...] = jnp.zeros_like(st2_ref)

    st2_ref[0:1, :] += jnp.sum(ns, axis=0)[None, :]
    st2_ref[1:2, :] += jnp.sum(ns * ns, axis=0)[None, :]


def _final_body(atom_ref, ns_ref, wskip_ref, bskip_ref, a2_ref, b2_ref,
                out_ref):
    skip = jnp.dot(atom_ref[...], wskip_ref[...],
                   preferred_element_type=jnp.float32) + bskip_ref[...]
    out_ref[...] = _softplus(skip + ns_ref[...] * a2_ref[...] + b2_ref[...])


def kernel(atom_in_fea, nbr_fea, nbr_fea_idx, W_full, b_full, bn1_gamma,
           bn1_beta, bn2_gamma, bn2_beta, W_skip, b_skip):
    idxf = nbr_fea_idx.astype(jnp.int32).reshape(_N * _M)
    idxp = jnp.pad(idxf, (0, (_NW * _CPW - _NCHUNK) * _CH))

    w_self = W_full[:_D]
    w_nbr = W_full[_D:2 * _D]
    w_e = W_full[2 * _D:]
    b2d = b_full.reshape(1, 2 * _D)

    ag = _sc_gather(atom_in_fea, idxp)

    wspec = [
        pl.BlockSpec((_D, 2 * _D), lambda i: (0, 0)),
        pl.BlockSpec((_E, 2 * _D), lambda i: (0, 0)),
        pl.BlockSpec((_D, 2 * _D), lambda i: (0, 0)),
        pl.BlockSpec((1, 2 * _D), lambda i: (0, 0)),
    ]
    row_specs = [
        pl.BlockSpec((_BI, _D), lambda i: (i, 0)),
        pl.BlockSpec((_BI * _M, _D), lambda i: (i, 0)),
        pl.BlockSpec((_BI, _M, _E), lambda i: (i, 0, 0)),
    ] + wspec

    stats1 = pl.pallas_call(
        _moments_body,
        grid=(_GRID,),
        in_specs=row_specs,
        out_specs=pl.BlockSpec((8, 2 * _D), lambda i: (0, 0)),
        out_shape=jax.ShapeDtypeStruct((8, 2 * _D), jnp.float32),
        scratch_shapes=[
            pltpu.VMEM((_D, _D), jnp.float32),
            pltpu.VMEM((_E, _D), jnp.float32),
            pltpu.VMEM((_E, _E), jnp.float32),
        ],
    )(atom_in_fea, ag, nbr_fea, w_self, w_e, w_nbr, b2d)

    cnt1 = jnp.float32(_N * _M)
    mean1 = stats1[0] / cnt1
    var1 = stats1[1] / cnt1 - mean1 * mean1
    a1 = bn1_gamma / jnp.sqrt(var1 + _EPS)
    b1 = bn1_beta - mean1 * a1

    ns, stats2 = pl.pallas_call(
        _reduce_body,
        grid=(_GRID,),
        in_specs=row_specs,
        out_specs=[
            pl.BlockSpec((_BI, _D), lambda i: (i, 0)),
            pl.BlockSpec((8, _D), lambda i: (0, 0)),
        ],
        out_shape=[
            jax.ShapeDtypeStruct((_N, _D), jnp.float32),
            jax.ShapeDtypeStruct((8, _D), jnp.float32),
        ],
    )(atom_in_fea, ag, nbr_fea, w_self * a1, w_e * a1, w_nbr * a1,
      (b_full * a1 + b1).reshape(1, 2 * _D))

    cnt2 = jnp.float32(_N)
    mean2 = stats2[0] / cnt2
    var2 = stats2[1] / cnt2 - mean2 * mean2
    a2 = bn2_gamma / jnp.sqrt(var2 + _EPS)
    b2 = bn2_beta - mean2 * a2

    out = pl.pallas_call(
        _final_body,
        grid=(_GRID,),
        in_specs=[
            pl.BlockSpec((_BI, _D), lambda i: (i, 0)),
            pl.BlockSpec((_BI, _D), lambda i: (i, 0)),
            pl.BlockSpec((_D, _D), lambda i: (0, 0)),
            pl.BlockSpec((1, _D), lambda i: (0, 0)),
            pl.BlockSpec((1, _D), lambda i: (0, 0)),
            pl.BlockSpec((1, _D), lambda i: (0, 0)),
        ],
        out_specs=pl.BlockSpec((_BI, _D), lambda i: (i, 0)),
        out_shape=jax.ShapeDtypeStruct((_N, _D), jnp.float32),
    )(atom_in_fea, ns, W_skip, b_skip.reshape(1, -1),
      a2.reshape(1, -1), b2.reshape(1, -1))

    return out


# speedup vs baseline: 1.0947x; 1.0005x over previous
import jax
import jax.numpy as jnp
from jax import lax
from jax.experimental import pallas as pl
from jax.experimental.pallas import tpu as pltpu
from jax.experimental.pallas import tpu_sc as plsc

_N = 10000
_M = 32
_D = 128
_E = 16
_CH = 128
_NCHUNK = (_N * _M) // _CH
_NW = 32
_CPW = 80
_NBUF = 4
_BI = 400
_GRID = _N // _BI
_EPS = 1e-5


def _softplus(x):
    return jnp.maximum(x, 0.0) + jnp.log1p(jnp.exp(-jnp.abs(x)))


def _dott(a, b):
    return lax.dot_general(a, b, (((0,), (0,)), ((), ())),
                           preferred_element_type=jnp.float32)


def _sc_gather_body(table_hbm, idx_hbm, out_hbm, idx_v, rows_v, gsem, wsem):
    wid = lax.axis_index("s") * 2 + lax.axis_index("c")
    base = wid * _CPW
    pltpu.sync_copy(idx_hbm.at[pl.ds(base * _CH, _CPW * _CH)], idx_v)

    def _wb_drain():
        pltpu.make_async_copy(
            rows_v.at[0], out_hbm.at[pl.ds(0, _CH)], wsem).wait()

    def _gather_fire(j, b):
        pltpu.async_copy(
            table_hbm.at[idx_v.at[pl.ds(j * _CH, _CH)]], rows_v.at[b], gsem)

    def _gather_wait(j, b):
        pltpu.make_async_copy(
            table_hbm.at[idx_v.at[pl.ds(j * _CH, _CH)]], rows_v.at[b],
            gsem).wait()

    def _wb_fire(b, c):
        pltpu.async_copy(rows_v.at[b], out_hbm.at[pl.ds(c * _CH, _CH)], wsem)

    ngrp = _CPW // _NBUF

    def body(g, carry):
        for b in range(_NBUF):
            j = g * _NBUF + b
            c = base + j

            @pl.when((g > 0) & (c - _NBUF < _NCHUNK))
            def _():
                _wb_drain()

            @pl.when(c < _NCHUNK)
            def _():
                _gather_fire(j, b)

            prev_ok = (c - 1 < _NCHUNK)
            if b == 0:
                prev_ok = (g > 0) & prev_ok

            @pl.when(prev_ok)
            def _():
                bp = (b - 1) % _NBUF
                _gather_wait(j - 1, bp)
                _wb_fire(bp, c - 1)

        return carry

    lax.fori_loop(0, ngrp, body, 0)

    last = _CPW - 1

    @pl.when(base + last < _NCHUNK)
    def _():
        _gather_wait(last, last % _NBUF)
        _wb_fire(last % _NBUF, base + last)

    for j in range(_CPW - _NBUF, _CPW):
        @pl.when(base + j < _NCHUNK)
        def _():
            _wb_drain()


def _sc_gather(table, idxp):
    call = pl.kernel(
        _sc_gather_body,
        out_type=jax.ShapeDtypeStruct((_N * _M, _D), jnp.float32),
        mesh=plsc.VectorSubcoreMesh(core_axis_name="c", subcore_axis_name="s"),
        scratch_types=[
            pltpu.VMEM((_CPW * _CH,), jnp.int32),
            pltpu.VMEM((_NBUF, _CH, _D), jnp.float32),
            pltpu.SemaphoreType.DMA,
            pltpu.SemaphoreType.DMA,
        ],
    )
    return call(table, idxp)


def _moments_body(atom_ref, ag_ref, nbr_ref, wself_ref, we_ref, wnbr_ref,
                  b_ref, out_ref, yy, ab, zz):
    i = pl.program_id(0)

    @pl.when(i == 0)
    def _():
        out_ref[...] = jnp.zeros_like(out_ref)
        yy[...] = jnp.zeros_like(yy)
        ab[...] = jnp.zeros_like(ab)
        zz[...] = jnp.zeros_like(zz)

    ag = ag_ref[...]
    nbr3 = nbr_ref[...]
    nbrf = nbr3.reshape(_BI * _M, _E)

    agb = ag.astype(jnp.bfloat16)
    nbrb = nbrf.astype(jnp.bfloat16)
    yy[...] += _dott(agb, agb)
    ab[...] += _dott(nbrb, agb)
    zz[...] += _dott(nbrb, nbrb)

    agsum = jnp.sum(ag.reshape(_BI, _M, _D), axis=1)
    nbrsum = jnp.sum(nbr3, axis=1)

    sp = jnp.dot(atom_ref[...], wself_ref[...],
                 preferred_element_type=jnp.float32) + b_ref[...]
    q = jnp.dot(agsum, wnbr_ref[...], preferred_element_type=jnp.float32)
    esum = jnp.dot(nbrsum, we_ref[...], preferred_element_type=jnp.float32)

    mf = jnp.float32(_M)
    t1 = jnp.sum(mf * sp + esum + q, axis=0)
    t2 = jnp.sum(mf * sp * sp + 2.0 * sp * (esum + q), axis=0)
    out_ref[0:1, :] += t1[None, :]
    out_ref[1:2, :] += t2[None, :]

    @pl.when(i == _GRID - 1)
    def _():
        wnbr = wnbr_ref[...]
        we = we_ref[...]
        g2 = jnp.sum(wnbr * jnp.dot(yy[...], wnbr,
                                    preferred_element_type=jnp.float32), 0)
        e2 = jnp.sum(we * jnp.dot(zz[...], we,
                                  preferred_element_type=jnp.float32), 0)
        eg = jnp.sum(we * jnp.dot(ab[...], wnbr,
                                  preferred_element_type=jnp.float32), 0)
        out_ref[1:2, :] += (g2 + e2 + 2.0 * eg)[None, :]


def _reduce_body(atom_ref, ag_ref, nbr_ref, wselfs_ref, wes_ref, wnbrs_ref,
                 bs_ref, ns_ref, st2_ref):
    s = jnp.dot(atom_ref[...], wselfs_ref[...],
                preferred_element_type=jnp.float32) + bs_ref[...]
    g = jnp.dot(ag_ref[...], wnbrs_ref[...], preferred_element_type=jnp.float32)
    e = jnp.dot(nbr_ref[...].reshape(_BI * _M, _E), wes_ref[...],
                preferred_element_type=jnp.float32)
    h = (g + e).reshape(_BI, _M, 2 * _D) + s[:, None, :]
    filt = jax.nn.sigmoid(h[:, :, :_D])
    core = _softplus(h[:, :, _D:])
    ns = jnp.sum(filt * core, axis=1)
    ns_ref[...] = ns

    @pl.when(pl.program_id(0) == 0)
    def _():
        st2_ref[...] = jnp.zeros_like(st2_ref)

    st2_ref[0:1, :] += jnp.sum(ns, axis=0)[None, :]
    st2_ref[1:2, :] += jnp.sum(ns * ns, axis=0)[None, :]


def _final_body(atom_ref, ns_ref, wskip_ref, bskip_ref, a2_ref, b2_ref,
                out_ref):
    skip = jnp.dot(atom_ref[...], wskip_ref[...],
                   preferred_element_type=jnp.float32) + bskip_ref[...]
    out_ref[...] = _softplus(skip + ns_ref[...] * a2_ref[...] + b2_ref[...])


def kernel(atom_in_fea, nbr_fea, nbr_fea_idx, W_full, b_full, bn1_gamma,
           bn1_beta, bn2_gamma, bn2_beta, W_skip, b_skip):
    idxf = nbr_fea_idx.astype(jnp.int32).reshape(_N * _M)
    idxp = jnp.pad(idxf, (0, (_NW * _CPW - _NCHUNK) * _CH))

    w_self = W_full[:_D]
    w_nbr = W_full[_D:2 * _D]
    w_e = W_full[2 * _D:]
    b2d = b_full.reshape(1, 2 * _D)

    ag = _sc_gather(atom_in_fea, idxp)

    wspec = [
        pl.BlockSpec((_D, 2 * _D), lambda i: (0, 0)),
        pl.BlockSpec((_E, 2 * _D), lambda i: (0, 0)),
        pl.BlockSpec((_D, 2 * _D), lambda i: (0, 0)),
        pl.BlockSpec((1, 2 * _D), lambda i: (0, 0)),
    ]
    row_specs = [
        pl.BlockSpec((_BI, _D), lambda i: (i, 0)),
        pl.BlockSpec((_BI * _M, _D), lambda i: (i, 0)),
        pl.BlockSpec((_BI, _M, _E), lambda i: (i, 0, 0)),
    ] + wspec

    stats1 = pl.pallas_call(
        _moments_body,
        grid=(_GRID,),
        in_specs=row_specs,
        out_specs=pl.BlockSpec((8, 2 * _D), lambda i: (0, 0)),
        out_shape=jax.ShapeDtypeStruct((8, 2 * _D), jnp.float32),
        scratch_shapes=[
            pltpu.VMEM((_D, _D), jnp.float32),
            pltpu.VMEM((_E, _D), jnp.float32),
            pltpu.VMEM((_E, _E), jnp.float32),
        ],
    )(atom_in_fea, ag, nbr_fea, w_self, w_e, w_nbr, b2d)

    cnt1 = jnp.float32(_N * _M)
    mean1 = stats1[0] / cnt1
    var1 = stats1[1] / cnt1 - mean1 * mean1
    a1 = bn1_gamma / jnp.sqrt(var1 + _EPS)
    b1 = bn1_beta - mean1 * a1

    ns, stats2 = pl.pallas_call(
        _reduce_body,
        grid=(_GRID,),
        in_specs=row_specs,
        out_specs=[
            pl.BlockSpec((_BI, _D), lambda i: (i, 0)),
            pl.BlockSpec((8, _D), lambda i: (0, 0)),
        ],
        out_shape=[
            jax.ShapeDtypeStruct((_N, _D), jnp.float32),
            jax.ShapeDtypeStruct((8, _D), jnp.float32),
        ],
    )(atom_in_fea, ag, nbr_fea, w_self * a1, w_e * a1, w_nbr * a1,
      (b_full * a1 + b1).reshape(1, 2 * _D))

    cnt2 = jnp.float32(_N)
    mean2 = stats2[0] / cnt2
    var2 = stats2[1] / cnt2 - mean2 * mean2
    a2 = bn2_gamma / jnp.sqrt(var2 + _EPS)
    b2 = bn2_beta - mean2 * a2

    out = pl.pallas_call(
        _final_body,
        grid=(_GRID,),
        in_specs=[
            pl.BlockSpec((_BI, _D), lambda i: (i, 0)),
            pl.BlockSpec((_BI, _D), lambda i: (i, 0)),
            pl.BlockSpec((_D, _D), lambda i: (0, 0)),
            pl.BlockSpec((1, _D), lambda i: (0, 0)),
            pl.BlockSpec((1, _D), lambda i: (0, 0)),
            pl.BlockSpec((1, _D), lambda i: (0, 0)),
        ],
        out_specs=pl.BlockSpec((_BI, _D), lambda i: (i, 0)),
        out_shape=jax.ShapeDtypeStruct((_N, _D), jnp.float32),
    )(atom_in_fea, ns, W_skip, b_skip.reshape(1, -1),
      a2.reshape(1, -1), b2.reshape(1, -1))

    return out
